# baseline (speedup 1.0000x reference)
import functools

import jax
import jax.numpy as jnp
from jax import lax
from jax.experimental import pallas as pl
from jax.experimental.pallas import tpu as pltpu
from jax.experimental.pallas import tpu_sc as plsc

NBUF = 4


@functools.lru_cache(maxsize=None)
def _build_sc_embed(BL, L, D):
    info = plsc.get_sparse_core_info()
    NC, NS = info.num_cores, info.num_subcores
    NW = NC * NS
    CHUNK = L // 2
    assert L % 2 == 0 and D % 16 == 0
    assert BL % (NW * L) == 0
    per_w = BL // NW
    n_chunks = per_w // CHUNK
    assert n_chunks % NBUF == 0 and n_chunks >= 2 * NBUF
    n_quads = n_chunks // NBUF
    mesh = plsc.VectorSubcoreMesh(core_axis_name="c", subcore_axis_name="s")

    @functools.partial(
        pl.kernel,
        mesh=mesh,
        compiler_params=pltpu.CompilerParams(use_tc_tiling_on_sc=False),
        out_type=jax.ShapeDtypeStruct((BL, D), jnp.float32),
        scratch_types=(
            [pltpu.VMEM((n_chunks, CHUNK), jnp.int32),
             pltpu.VMEM((L, D), jnp.float32)]
            + [pltpu.VMEM((CHUNK, D), jnp.float32) for _ in range(NBUF)]
            + [pltpu.SemaphoreType.DMA for _ in range(2 * NBUF)]
        ),
    )
    def embed(x_hbm, tok_hbm, pos_hbm, out_hbm, idx_v, pos_v, *bufs_and_sems):
        rows = bufs_and_sems[:NBUF]
        gsem = bufs_and_sems[NBUF:2 * NBUF]
        ssem = bufs_and_sems[2 * NBUF:]
        wid = lax.axis_index("s") * NC + lax.axis_index("c")
        base = wid * per_w

        pltpu.sync_copy(x_hbm.at[wid], idx_v)
        pltpu.sync_copy(pos_hbm, pos_v)

        def start_gather(g, b):
            pltpu.make_async_copy(
                tok_hbm.at[idx_v.at[g]], rows[b], gsem[b]).start()

        def wait_gather(g, b):
            pltpu.make_async_copy(
                tok_hbm.at[idx_v.at[g]], rows[b], gsem[b]).wait()

        def start_store(g, b):
            pltpu.make_async_copy(
                rows[b], out_hbm.at[pl.ds(base + g * CHUNK, CHUNK)],
                ssem[b]).start()

        def wait_store(b):
            pltpu.make_async_copy(
                rows[b], out_hbm.at[pl.ds(base, CHUNK)], ssem[b]).wait()

        def add_pos(b, half):
            prow = half * CHUNK

            def row_body(i, c):
                for q in range(D // 16):
                    sl = pl.ds(q * 16, 16)
                    plsc.addupdate(rows[b].at[i, sl], pos_v[prow + i, sl])
                return c

            lax.fori_loop(0, CHUNK, row_body, 0, unroll=4)

        def chunk_body(g, b, half, prefetch, prefetch_wait):
            wait_gather(g, b)
            add_pos(b, half)
            start_store(g, b)
            if prefetch:
                nb = (b + NBUF - 1) % NBUF
                if prefetch_wait:
                    wait_store(nb)
                start_gather(g + NBUF - 1, nb)

        for b in range(NBUF - 1):
            start_gather(b, b)
        chunk_body(0, 0, 0, True, False)
        for b in range(1, NBUF):
            chunk_body(b, b, b % 2, True, True)

        def quad(p, c):
            g0 = p * NBUF
            for b in range(NBUF):
                chunk_body(g0 + b, b, b % 2, True, True)
            return c

        lax.fori_loop(1, n_quads - 1, quad, 0)

        g0 = (n_quads - 1) * NBUF
        chunk_body(g0, 0, 0, True, True)
        for b in range(1, NBUF):
            chunk_body(g0 + b, b, b % 2, False, False)
        for b in range(NBUF):
            wait_store(b)

    return embed


def kernel(x, token_table, pos_table):
    B, L = x.shape
    D = token_table.shape[1]
    BL = B * L
    info = plsc.get_sparse_core_info()
    NW = info.num_cores * info.num_subcores
    CHUNK = L // 2
    x_r = x.astype(jnp.int32).reshape(NW, BL // (NW * CHUNK), CHUNK)
    out = _build_sc_embed(BL, L, D)(x_r, token_table, pos_table)
    return out.reshape(B, L, D)
